# baseline (speedup 1.0000x reference)
import functools

import jax
import jax.numpy as jnp
from jax import lax
from jax.experimental import pallas as pl
from jax.experimental.pallas import tpu as pltpu
from jax.experimental.pallas import tpu_sc as plsc


def _leaky(h):
    return jnp.where(h > 0, h, 0.01 * h)


def _scores_body(x_ref, w1_ref, b1_ref, w2_ref, b2_ref, w3_ref, b3_ref, s_ref):
    nb = x_ref.shape[0]
    for i in range(nb):
        xb = x_ref[i]
        h1 = jnp.dot(w1_ref[...], xb, preferred_element_type=jnp.float32) + b1_ref[...]
        h1 = _leaky(h1)
        h2 = jnp.dot(w2_ref[...], h1, preferred_element_type=jnp.float32) + b2_ref[...]
        h2 = _leaky(h2)
        s = jnp.dot(w3_ref[...], h2, preferred_element_type=jnp.float32) + b3_ref[...]
        s_ref[i] = jnp.tanh(s)


_MASKED = -2.0


def _perm(v, idx):
    return lax.gather(
        v, idx[:, None],
        lax.GatherDimensionNumbers(offset_dims=(), collapsed_slice_dims=(0,),
                                   start_index_map=(0,)),
        slice_sizes=(1,), mode=lax.GatherScatterMode.PROMISE_IN_BOUNDS)


def _allmax(v, lanes):
    for sh in (8, 4, 2, 1):
        v = jnp.maximum(v, _perm(v, lanes ^ sh))
    return v


def _allmin(v, lanes):
    for sh in (8, 4, 2, 1):
        v = jnp.minimum(v, _perm(v, lanes ^ sh))
    return v


def _cmp_desc(av, ai, bv, bi):
    return jnp.logical_or(av > bv, jnp.logical_and(av == bv, ai < bi))


def _topk_sc_body(k, n, scores_hbm, out_hbm, s_v, idx_v, nb_v, nb_i,
                  sh_v, sh_i, sem):
    del sem
    core = lax.axis_index("c")
    sid = lax.axis_index("s")
    nh = n // 2
    row = core * 8 + sid // 2
    half = lax.rem(sid, 2)
    base = half * nh
    nv = nh // 16

    pltpu.sync_copy(scores_hbm.at[pl.ds(row * n + base, nh)], s_v)
    lanes = lax.iota(jnp.int32, 16)
    masked_v = jnp.full((16,), _MASKED, jnp.float32)

    def build(j, carry):
        m, jv = carry
        v = s_v[pl.ds(j * 16, 16)]
        better = v > m
        return jnp.where(better, v, m), jnp.where(better, j, jv)

    m, jv = lax.fori_loop(0, nv, build,
                          (masked_v, jnp.zeros((16,), jnp.int32)), unroll=8)

    def select(it, carry):
        m, jv, acc, vacc = carry
        top = _allmax(m, lanes)
        pos_vec = jv * 16 + lanes
        cand = jnp.where(m == top, pos_vec, jnp.int32(nh))
        posv = _allmin(cand, lanes)
        pos = posv[0]
        lane = lax.rem(pos, 16)
        jbase = pos - lane
        at_lane = lanes == lane
        v = s_v[pl.ds(jbase, 16)]
        s_v[pl.ds(jbase, 16)] = jnp.where(at_lane, masked_v, v)
        hit = lanes == lax.rem(it, 16)
        t = it // 16
        acc = tuple(
            jnp.where(jnp.logical_and(t == tt, hit), pos, acc[tt])
            for tt in range(k // 16))
        vacc = tuple(
            jnp.where(jnp.logical_and(t == tt, hit), top, vacc[tt])
            for tt in range(k // 16))
        best_v = masked_v
        gvs = []
        for g in range(nv // 16):
            gidx = (g * 16 + lanes) * 16 + lane
            gv = plsc.load_gather(s_v, [gidx])
            gvs.append(gv)
            best_v = jnp.maximum(best_v, gv)
        new_m = _allmax(best_v, lanes)
        best_j = jnp.full((16,), jnp.int32(nh), jnp.int32)
        for g in range(nv // 16):
            best_j = jnp.minimum(
                best_j,
                jnp.where(gvs[g] == new_m, g * 16 + lanes, jnp.int32(nh)))
        best_j = _allmin(best_j, lanes)
        m = jnp.where(at_lane, new_m, m)
        jv = jnp.where(at_lane, best_j, jv)
        return m, jv, acc, vacc

    acc0 = tuple(jnp.zeros((16,), jnp.int32) for _ in range(k // 16))
    vacc0 = tuple(jnp.full((16,), _MASKED, jnp.float32)
                  for _ in range(k // 16))
    _, _, acc, vacc = lax.fori_loop(0, k, select, (m, jv, acc0, vacc0),
                                    unroll=2)
    acc = tuple(a + base for a in acc)

    for tt in range(k // 16):
        idx_v[pl.ds(tt * 16, 16)] = acc[tt]
        nb_v[pl.ds(tt * 16, 16)] = vacc[tt]
    pltpu.sync_copy(idx_v, sh_i.at[pl.ds(sid * k, k)])
    pltpu.sync_copy(nb_v, sh_v.at[pl.ds(sid * k, k)])
    plsc.subcore_barrier()

    @pl.when(half == 0)
    def _():
        pltpu.sync_copy(sh_v.at[pl.ds((sid + 1) * k, k)], nb_v)
        pltpu.sync_copy(sh_i.at[pl.ds((sid + 1) * k, k)], nb_i)
        Av, Ai = list(vacc), list(acc)
        Bv = [nb_v[pl.ds(tt * 16, 16)] for tt in range(4)]
        Bi = [nb_i[pl.ds(tt * 16, 16)] for tt in range(4)]
        Lv, Li = [], []
        for tt in range(4):
            rbv = lax.rev(Bv[3 - tt], (0,))
            rbi = lax.rev(Bi[3 - tt], (0,))
            c = _cmp_desc(Av[tt], Ai[tt], rbv, rbi)
            Lv.append(jnp.where(c, Av[tt], rbv))
            Li.append(jnp.where(c, Ai[tt], rbi))
        for pairs in (((0, 2), (1, 3)), ((0, 1), (2, 3))):
            for (i, j) in pairs:
                c = _cmp_desc(Lv[i], Li[i], Lv[j], Li[j])
                hv = jnp.where(c, Lv[i], Lv[j])
                hi = jnp.where(c, Li[i], Li[j])
                lv = jnp.where(c, Lv[j], Lv[i])
                li = jnp.where(c, Li[j], Li[i])
                Lv[i], Li[i], Lv[j], Li[j] = hv, hi, lv, li
        for dd in (8, 4, 2, 1):
            for tt in range(4):
                pv = _perm(Lv[tt], lanes ^ dd)
                pi = _perm(Li[tt], lanes ^ dd)
                c = _cmp_desc(Lv[tt], Li[tt], pv, pi)
                takemax = (lanes & dd) == 0
                keep = jnp.logical_xor(c, jnp.logical_not(takemax))
                Lv[tt] = jnp.where(keep, Lv[tt], pv)
                Li[tt] = jnp.where(keep, Li[tt], pi)
        for tt in range(4):
            idx_v[pl.ds(tt * 16, 16)] = Li[tt]
        pltpu.sync_copy(idx_v, out_hbm.at[pl.ds(row * k, k)])


def kernel(x, K, W1, b1, W2, b2, W3, b3):
    B, C, N = x.shape
    k = W1.shape[0]
    NB = 2

    scores = pl.pallas_call(
        _scores_body,
        grid=(B // NB,),
        in_specs=[
            pl.BlockSpec((NB, C, N), lambda b: (b, 0, 0)),
            pl.BlockSpec(W1.shape, lambda b: (0, 0)),
            pl.BlockSpec((W1.shape[0], 1), lambda b: (0, 0)),
            pl.BlockSpec(W2.shape, lambda b: (0, 0)),
            pl.BlockSpec((W2.shape[0], 1), lambda b: (0, 0)),
            pl.BlockSpec(W3.shape, lambda b: (0, 0)),
            pl.BlockSpec((1, 1), lambda b: (0, 0)),
        ],
        out_specs=pl.BlockSpec((NB, 1, N), lambda b: (b, 0, 0)),
        out_shape=jax.ShapeDtypeStruct((B, 1, N), jnp.float32),
    )(x, W1, b1.reshape(-1, 1), W2, b2.reshape(-1, 1), W3, b3.reshape(1, 1))

    topk = functools.partial(
        pl.kernel,
        out_type=jax.ShapeDtypeStruct((B * k,), jnp.int32),
        mesh=plsc.VectorSubcoreMesh(core_axis_name="c", subcore_axis_name="s"),
        scratch_types=[
            pltpu.VMEM((N // 2,), jnp.float32),
            pltpu.VMEM((k,), jnp.int32),
            pltpu.VMEM((k,), jnp.float32),
            pltpu.VMEM((k,), jnp.int32),
            pltpu.VMEM_SHARED((16 * k,), jnp.float32),
            pltpu.VMEM_SHARED((16 * k,), jnp.int32),
            pltpu.SemaphoreType.DMA,
        ],
        compiler_params=pltpu.CompilerParams(needs_layout_passes=False),
    )(functools.partial(_topk_sc_body, k, N))
    idx = topk(scores.reshape(-1))

    return idx


# speedup vs baseline: 1.0059x; 1.0059x over previous
import functools

import jax
import jax.numpy as jnp
from jax import lax
from jax.experimental import pallas as pl
from jax.experimental.pallas import tpu as pltpu
from jax.experimental.pallas import tpu_sc as plsc


def _leaky(h):
    return jnp.where(h > 0, h, 0.01 * h)


def _scores_body(x_ref, w1_ref, b1_ref, w2_ref, b2_ref, w3_ref, b3_ref, s_ref):
    nb = x_ref.shape[0]
    for i in range(nb):
        xb = x_ref[i]
        h1 = jnp.dot(w1_ref[...], xb, preferred_element_type=jnp.float32) + b1_ref[...]
        h1 = _leaky(h1)
        h2 = jnp.dot(w2_ref[...], h1, preferred_element_type=jnp.float32) + b2_ref[...]
        h2 = _leaky(h2)
        s = jnp.dot(w3_ref[...], h2, preferred_element_type=jnp.float32) + b3_ref[...]
        s_ref[i] = jnp.tanh(s)


_MASKED = -2.0


def _perm(v, idx):
    return lax.gather(
        v, idx[:, None],
        lax.GatherDimensionNumbers(offset_dims=(), collapsed_slice_dims=(0,),
                                   start_index_map=(0,)),
        slice_sizes=(1,), mode=lax.GatherScatterMode.PROMISE_IN_BOUNDS)


def _allmax(v, lanes):
    for sh in (8, 4, 2, 1):
        v = jnp.maximum(v, _perm(v, lanes ^ sh))
    return v


def _allmin(v, lanes):
    for sh in (8, 4, 2, 1):
        v = jnp.minimum(v, _perm(v, lanes ^ sh))
    return v


def _cmp_desc(av, ai, bv, bi):
    return jnp.logical_or(av > bv, jnp.logical_and(av == bv, ai < bi))


def _topk_sc_body(k, n, scores_hbm, out_hbm, s_v, idx_v, nb_v, nb_i,
                  sh_v, sh_i, sem):
    del sem
    core = lax.axis_index("c")
    sid = lax.axis_index("s")
    nh = n // 2
    row = core * 8 + sid // 2
    half = lax.rem(sid, 2)
    base = half * nh
    nv = nh // 16

    pltpu.sync_copy(scores_hbm.at[pl.ds(row * n + base, nh)], s_v)
    lanes = lax.iota(jnp.int32, 16)
    masked_v = jnp.full((16,), _MASKED, jnp.float32)

    def build(j, carry):
        m, jv = carry
        v = s_v[pl.ds(j * 16, 16)]
        better = v > m
        return jnp.where(better, v, m), jnp.where(better, j, jv)

    m, jv = lax.fori_loop(0, nv, build,
                          (masked_v, jnp.zeros((16,), jnp.int32)), unroll=8)

    def select(it, carry):
        m, jv, acc, vacc = carry
        top = _allmax(m, lanes)
        pos_vec = jv * 16 + lanes
        cand = jnp.where(m == top, pos_vec, jnp.int32(nh))
        posv = _allmin(cand, lanes)
        pos = posv[0]
        lane = lax.rem(pos, 16)
        jbase = pos - lane
        at_lane = lanes == lane
        v = s_v[pl.ds(jbase, 16)]
        s_v[pl.ds(jbase, 16)] = jnp.where(at_lane, masked_v, v)
        hit = lanes == lax.rem(it, 16)
        t = it // 16
        acc = tuple(
            jnp.where(jnp.logical_and(t == tt, hit), pos, acc[tt])
            for tt in range(k // 16))
        vacc = tuple(
            jnp.where(jnp.logical_and(t == tt, hit), top, vacc[tt])
            for tt in range(k // 16))
        best_v = masked_v
        gvs = []
        for g in range(nv // 16):
            gidx = (g * 16 + lanes) * 16 + lane
            gv = plsc.load_gather(s_v, [gidx])
            gvs.append(gv)
            best_v = jnp.maximum(best_v, gv)
        new_m = _allmax(best_v, lanes)
        best_j = jnp.full((16,), jnp.int32(nh), jnp.int32)
        for g in range(nv // 16):
            best_j = jnp.minimum(
                best_j,
                jnp.where(gvs[g] == new_m, g * 16 + lanes, jnp.int32(nh)))
        best_j = _allmin(best_j, lanes)
        m = jnp.where(at_lane, new_m, m)
        jv = jnp.where(at_lane, best_j, jv)
        return m, jv, acc, vacc

    acc0 = tuple(jnp.zeros((16,), jnp.int32) for _ in range(k // 16))
    vacc0 = tuple(jnp.full((16,), _MASKED, jnp.float32)
                  for _ in range(k // 16))
    _, _, acc, vacc = lax.fori_loop(0, k, select, (m, jv, acc0, vacc0))
    acc = tuple(a + base for a in acc)

    for tt in range(k // 16):
        idx_v[pl.ds(tt * 16, 16)] = acc[tt]
        nb_v[pl.ds(tt * 16, 16)] = vacc[tt]
    pltpu.sync_copy(idx_v, sh_i.at[pl.ds(sid * k, k)])
    pltpu.sync_copy(nb_v, sh_v.at[pl.ds(sid * k, k)])
    plsc.subcore_barrier()

    @pl.when(half == 0)
    def _():
        pltpu.sync_copy(sh_v.at[pl.ds((sid + 1) * k, k)], nb_v)
        pltpu.sync_copy(sh_i.at[pl.ds((sid + 1) * k, k)], nb_i)
        Av, Ai = list(vacc), list(acc)
        Bv = [nb_v[pl.ds(tt * 16, 16)] for tt in range(4)]
        Bi = [nb_i[pl.ds(tt * 16, 16)] for tt in range(4)]
        Lv, Li = [], []
        for tt in range(4):
            rbv = lax.rev(Bv[3 - tt], (0,))
            rbi = lax.rev(Bi[3 - tt], (0,))
            c = _cmp_desc(Av[tt], Ai[tt], rbv, rbi)
            Lv.append(jnp.where(c, Av[tt], rbv))
            Li.append(jnp.where(c, Ai[tt], rbi))
        for pairs in (((0, 2), (1, 3)), ((0, 1), (2, 3))):
            for (i, j) in pairs:
                c = _cmp_desc(Lv[i], Li[i], Lv[j], Li[j])
                hv = jnp.where(c, Lv[i], Lv[j])
                hi = jnp.where(c, Li[i], Li[j])
                lv = jnp.where(c, Lv[j], Lv[i])
                li = jnp.where(c, Li[j], Li[i])
                Lv[i], Li[i], Lv[j], Li[j] = hv, hi, lv, li
        for dd in (8, 4, 2, 1):
            for tt in range(4):
                pv = _perm(Lv[tt], lanes ^ dd)
                pi = _perm(Li[tt], lanes ^ dd)
                c = _cmp_desc(Lv[tt], Li[tt], pv, pi)
                takemax = (lanes & dd) == 0
                keep = jnp.logical_xor(c, jnp.logical_not(takemax))
                Lv[tt] = jnp.where(keep, Lv[tt], pv)
                Li[tt] = jnp.where(keep, Li[tt], pi)
        for tt in range(4):
            idx_v[pl.ds(tt * 16, 16)] = Li[tt]
        pltpu.sync_copy(idx_v, out_hbm.at[pl.ds(row * k, k)])


def kernel(x, K, W1, b1, W2, b2, W3, b3):
    B, C, N = x.shape
    k = W1.shape[0]
    NB = 2

    scores = pl.pallas_call(
        _scores_body,
        grid=(B // NB,),
        in_specs=[
            pl.BlockSpec((NB, C, N), lambda b: (b, 0, 0)),
            pl.BlockSpec(W1.shape, lambda b: (0, 0)),
            pl.BlockSpec((W1.shape[0], 1), lambda b: (0, 0)),
            pl.BlockSpec(W2.shape, lambda b: (0, 0)),
            pl.BlockSpec((W2.shape[0], 1), lambda b: (0, 0)),
            pl.BlockSpec(W3.shape, lambda b: (0, 0)),
            pl.BlockSpec((1, 1), lambda b: (0, 0)),
        ],
        out_specs=pl.BlockSpec((NB, 1, N), lambda b: (b, 0, 0)),
        out_shape=jax.ShapeDtypeStruct((B, 1, N), jnp.float32),
    )(x, W1, b1.reshape(-1, 1), W2, b2.reshape(-1, 1), W3, b3.reshape(1, 1))

    topk = functools.partial(
        pl.kernel,
        out_type=jax.ShapeDtypeStruct((B * k,), jnp.int32),
        mesh=plsc.VectorSubcoreMesh(core_axis_name="c", subcore_axis_name="s"),
        scratch_types=[
            pltpu.VMEM((N // 2,), jnp.float32),
            pltpu.VMEM((k,), jnp.int32),
            pltpu.VMEM((k,), jnp.float32),
            pltpu.VMEM((k,), jnp.int32),
            pltpu.VMEM_SHARED((16 * k,), jnp.float32),
            pltpu.VMEM_SHARED((16 * k,), jnp.int32),
            pltpu.SemaphoreType.DMA,
        ],
        compiler_params=pltpu.CompilerParams(needs_layout_passes=False),
    )(functools.partial(_topk_sc_body, k, N))
    idx = topk(scores.reshape(-1))

    return idx
